# Initial kernel scaffold; baseline (speedup 1.0000x reference)
import jax
import jax.numpy as jnp
from jax.experimental import pallas as pl


def kernel(x, edge_index, batch, W1, b1, W2, b2, W3, b3, Wl1, bl1, Wl2, bl2, Wl3, bl3):
    raise NotImplementedError("write your pallas kernel here")


# speedup vs baseline: 14.7540x; 14.7540x over previous
import functools

import jax
import jax.numpy as jnp
from jax import lax
from jax.experimental import pallas as pl
from jax.experimental.pallas import tpu as pltpu
from jax.experimental.pallas import tpu_sc as plsc

N = 10000
E = 640000
F = 128
G = 128
NC = 2
NS = 16
NW = NC * NS
EPW = E // NW
CH = 128
NFULL = EPW // CH
REM = EPW - NFULL * CH
DEGW = 16
DEGP = 10240
DSTR = DEGP // NS
NPAD = 10240
RSTR = NPAD // NS

_mesh = plsc.VectorSubcoreMesh(core_axis_name="c", subcore_axis_name="s")


@functools.partial(
    pl.kernel,
    mesh=_mesh,
    out_type=jax.ShapeDtypeStruct((NC, DEGP, DEGW), jnp.float32),
    scratch_types=[
        pltpu.VMEM((CH,), jnp.int32),
        pltpu.VMEM((REM,), jnp.int32),
        pltpu.VMEM((CH, DEGW), jnp.float32),
        pltpu.VMEM_SHARED((DEGP, DEGW), jnp.float32),
    ],
)
def _deg_kernel(dst, ones_hbm, zdeg, out, didx, didx_r, ones_v, acc):
    c = lax.axis_index("c")
    s = lax.axis_index("s")
    pltpu.sync_copy(ones_hbm, ones_v)
    pltpu.sync_copy(zdeg.at[pl.ds(s * DSTR, DSTR)], acc.at[pl.ds(s * DSTR, DSTR)])
    plsc.subcore_barrier()
    base = (s * NC + c) * EPW

    def body(g_, _):
        off = base + g_ * CH
        pltpu.sync_copy(dst.at[pl.ds(off, CH)], didx)
        pltpu.sync_copy(ones_v, acc.at[didx], add=True)
        return 0

    lax.fori_loop(0, NFULL, body, 0)
    off = base + NFULL * CH
    pltpu.sync_copy(dst.at[pl.ds(off, REM)], didx_r)
    pltpu.sync_copy(ones_v.at[pl.ds(0, REM)], acc.at[didx_r], add=True)
    plsc.subcore_barrier()
    pltpu.sync_copy(acc.at[pl.ds(s * DSTR, DSTR)], out.at[c, pl.ds(s * DSTR, DSTR)])


@functools.partial(
    pl.kernel,
    mesh=_mesh,
    out_type=jax.ShapeDtypeStruct((NC, NPAD, F), jnp.float32),
    scratch_types=[
        pltpu.VMEM((CH,), jnp.int32),
        pltpu.VMEM((CH,), jnp.int32),
        pltpu.VMEM((REM,), jnp.int32),
        pltpu.VMEM((REM,), jnp.int32),
        pltpu.VMEM((CH, F), jnp.float32),
        pltpu.VMEM((REM, F), jnp.float32),
        pltpu.VMEM_SHARED((NPAD, F), jnp.float32),
        pltpu.SemaphoreType.DMA,
    ],
)
def _agg_kernel(hp, src, dst, znode, out, sidx, didx, sidx_r, didx_r, rows, rows_r, acc, sem):
    c = lax.axis_index("c")
    s = lax.axis_index("s")
    pltpu.sync_copy(znode.at[pl.ds(s * RSTR, RSTR)], acc.at[pl.ds(s * RSTR, RSTR)])
    plsc.subcore_barrier()
    base = (s * NC + c) * EPW

    def body(g_, _):
        off = base + g_ * CH
        pltpu.sync_copy(src.at[pl.ds(off, CH)], sidx)
        pltpu.sync_copy(dst.at[pl.ds(off, CH)], didx)
        pltpu.async_copy(hp.at[sidx], rows, sem).wait()
        pltpu.sync_copy(rows, acc.at[didx], add=True)
        return 0

    lax.fori_loop(0, NFULL, body, 0)
    off = base + NFULL * CH
    pltpu.sync_copy(src.at[pl.ds(off, REM)], sidx_r)
    pltpu.sync_copy(dst.at[pl.ds(off, REM)], didx_r)
    pltpu.async_copy(hp.at[sidx_r], rows_r, sem).wait()
    pltpu.sync_copy(rows_r, acc.at[didx_r], add=True)
    plsc.subcore_barrier()
    pltpu.sync_copy(acc.at[pl.ds(s * RSTR, RSTR)], out.at[c, pl.ds(s * RSTR, RSTR)])


ROWS_BLK = 2000


def _tc1_body(degp_ref, x_ref, w_ref, out_ref):
    deg = degp_ref[0, :, 0:1] + degp_ref[1, :, 0:1] + 1.0
    dis = lax.rsqrt(deg)
    h = jnp.dot(x_ref[...], w_ref[...], preferred_element_type=jnp.float32)
    out_ref[...] = h * dis


_tc1 = pl.pallas_call(
    _tc1_body,
    grid=(N // ROWS_BLK,),
    in_specs=[
        pl.BlockSpec((NC, ROWS_BLK, DEGW), lambda r: (0, r, 0)),
        pl.BlockSpec((ROWS_BLK, F), lambda r: (r, 0)),
        pl.BlockSpec((F, F), lambda r: (0, 0)),
    ],
    out_specs=pl.BlockSpec((ROWS_BLK, F), lambda r: (r, 0)),
    out_shape=jax.ShapeDtypeStruct((N, F), jnp.float32),
)


def _tcl_body(degp_ref, p_ref, hp_ref, w_ref, b_ref, out_ref):
    dis = lax.rsqrt(degp_ref[0, :, 0:1] + degp_ref[1, :, 0:1] + 1.0)
    agg = p_ref[0] + p_ref[1] + hp_ref[...]
    xl = jnp.maximum(agg * dis + b_ref[...], 0.0)
    out_ref[...] = jnp.dot(xl, w_ref[...], preferred_element_type=jnp.float32) * dis


_tcl = pl.pallas_call(
    _tcl_body,
    grid=(N // ROWS_BLK,),
    in_specs=[
        pl.BlockSpec((NC, ROWS_BLK, DEGW), lambda r: (0, r, 0)),
        pl.BlockSpec((NC, ROWS_BLK, F), lambda r: (0, r, 0)),
        pl.BlockSpec((ROWS_BLK, F), lambda r: (r, 0)),
        pl.BlockSpec((F, F), lambda r: (0, 0)),
        pl.BlockSpec((1, F), lambda r: (0, 0)),
    ],
    out_specs=pl.BlockSpec((ROWS_BLK, F), lambda r: (r, 0)),
    out_shape=jax.ShapeDtypeStruct((N, F), jnp.float32),
)


def _tcf_body(degp_ref, p_ref, hp_ref, b_ref, out_ref):
    dis = lax.rsqrt(degp_ref[0, :, 0:1] + degp_ref[1, :, 0:1] + 1.0)
    agg = p_ref[0] + p_ref[1] + hp_ref[...]
    out_ref[...] = jnp.maximum(agg * dis + b_ref[...], 0.0)


_tcf = pl.pallas_call(
    _tcf_body,
    grid=(N // ROWS_BLK,),
    in_specs=[
        pl.BlockSpec((NC, ROWS_BLK, DEGW), lambda r: (0, r, 0)),
        pl.BlockSpec((NC, ROWS_BLK, F), lambda r: (0, r, 0)),
        pl.BlockSpec((ROWS_BLK, F), lambda r: (r, 0)),
        pl.BlockSpec((1, F), lambda r: (0, 0)),
    ],
    out_specs=pl.BlockSpec((ROWS_BLK, F), lambda r: (r, 0)),
    out_shape=jax.ShapeDtypeStruct((N, F), jnp.float32),
)


def _pool_body(x4_ref, bt_ref, wl1_ref, bl1_ref, wl2_ref, bl2_ref, wl3_ref, bl3_ref, out_ref):
    bt = bt_ref[...]
    gids = lax.broadcasted_iota(jnp.int32, (1, G), 1)
    oh = (bt == gids).astype(jnp.float32)
    dn = (((0,), (0,)), ((), ()))
    sums = lax.dot_general(oh, x4_ref[...], dn, preferred_element_type=jnp.float32)
    cnts = lax.dot_general(oh, jnp.ones((N, 1), jnp.float32), dn,
                           preferred_element_type=jnp.float32)
    pooled = sums / jnp.maximum(cnts, 1.0)
    h = jnp.maximum(jnp.dot(pooled, wl1_ref[...], preferred_element_type=jnp.float32)
                    + bl1_ref[...], 0.0)
    h = jnp.maximum(jnp.dot(h, wl2_ref[...], preferred_element_type=jnp.float32)
                    + bl2_ref[...], 0.0)
    out_ref[...] = jnp.dot(h, wl3_ref[...], preferred_element_type=jnp.float32) + bl3_ref[...]


_pool = pl.pallas_call(
    _pool_body,
    out_shape=jax.ShapeDtypeStruct((G, 1), jnp.float32),
)


def kernel(x, edge_index, batch, W1, b1, W2, b2, W3, b3, Wl1, bl1, Wl2, bl2, Wl3, bl3):
    ones16 = jnp.ones((CH, DEGW), jnp.float32)
    zdeg = jnp.zeros((DEGP, DEGW), jnp.float32)
    znode = jnp.zeros((NPAD, F), jnp.float32)

    src = edge_index[0]
    dst = edge_index[1]
    degp = _deg_kernel(dst, ones16, zdeg)
    h1p = _tc1(degp, x, W1)
    p1 = _agg_kernel(h1p, src, dst, znode)
    h2p = _tcl(degp, p1, h1p, W2, b1.reshape(1, F))
    p2 = _agg_kernel(h2p, src, dst, znode)
    h3p = _tcl(degp, p2, h2p, W3, b2.reshape(1, F))
    p3 = _agg_kernel(h3p, src, dst, znode)
    x4 = _tcf(degp, p3, h3p, b3.reshape(1, F))
    outg = _pool(x4, batch.reshape(N, 1), Wl1, bl1.reshape(1, F),
                 Wl2, bl2.reshape(1, F), Wl3, bl3.reshape(1, 1))
    return outg.reshape(-1)
